# baseline (speedup 1.0000x reference)
import functools

import jax
import jax.numpy as jnp
from jax import lax
from jax.experimental import pallas as pl
from jax.experimental.pallas import tpu as pltpu
from jax.experimental.pallas import tpu_sc as plsc

VOCAB = 1000000
EMB_DIM = 64
NUM_Y = 2
BATCH = 4096
HIST = 200

NC = 2
NS = 16
NW = NC * NS

TOTAL_VALS = BATCH * HIST * NUM_Y
VALS_PER_W = TOTAL_VALS // NW
CHUNK = 128
CHUNKS_PER_W = VALS_PER_W // CHUNK
INFLIGHT = 8
ROUNDS = CHUNKS_PER_W // INFLIGHT
BAGS_PER_W = BATCH // NW
ROW_VALS = HIST * NUM_Y
VREGS_PER_ROW = ROW_VALS // 16


def _proj_body(emb_ref, w_ref, out_ref):
    out_ref[...] = lax.dot_general(
        emb_ref[...], w_ref[...],
        (((1,), (1,)), ((), ())),
        preferred_element_type=jnp.float32,
    )


def _project_table(emb_weight, W):
    rows_per_blk = 8000
    grid = VOCAB // rows_per_blk
    return pl.pallas_call(
        _proj_body,
        grid=(grid,),
        in_specs=[
            pl.BlockSpec((rows_per_blk, EMB_DIM), lambda i: (i, 0)),
            pl.BlockSpec((NUM_Y, EMB_DIM), lambda i: (0, 0)),
        ],
        out_specs=pl.BlockSpec((rows_per_blk, NUM_Y), lambda i: (i, 0)),
        out_shape=jax.ShapeDtypeStruct((VOCAB, NUM_Y), jnp.float32),
    )(emb_weight, W)


def _sc_body(p_hbm, idx_hbm, bias_hbm, out_hbm, idx_v, g_v, o_v, b_v, sem):
    wid = lax.axis_index("s") * NC + lax.axis_index("c")
    pltpu.sync_copy(idx_hbm.at[pl.ds(wid * CHUNKS_PER_W, CHUNKS_PER_W)], idx_v)
    pltpu.sync_copy(bias_hbm, b_v)

    def round_fn(r, carry):
        base = r * INFLIGHT
        cps = []
        for j in range(INFLIGHT):
            row = base + j
            cp = pltpu.make_async_copy(
                p_hbm.at[idx_v.at[row]],
                g_v.at[pl.ds(row * CHUNK, CHUNK)],
                sem,
            )
            cp.start()
            cps.append(cp)
        for cp in cps:
            cp.wait()
        return carry

    lax.fori_loop(0, ROUNDS, round_fn, 0)

    lanes = lax.iota(jnp.int32, 16)
    perm8 = (lanes + 8) % 16
    perm4 = (lanes + 4) % 16
    perm2 = (lanes + 2) % 16
    bias = b_v[...]
    dnums = lax.GatherDimensionNumbers(
        offset_dims=(), collapsed_slice_dims=(0,), start_index_map=(0,))

    def lane_perm(x, perm):
        return lax.gather(x, perm[:, None], dnums, (1,),
                          mode=lax.GatherScatterMode.PROMISE_IN_BOUNDS)

    def per_group(g, carry):
        out_acc = jnp.zeros((16,), jnp.float32)
        for j in range(8):
            r = g * 8 + j

            def red(k, a):
                return a + g_v[pl.ds(r * ROW_VALS + k * 16, 16)]

            acc = lax.fori_loop(0, VREGS_PER_ROW, red,
                                jnp.zeros((16,), jnp.float32))
            acc = acc + lane_perm(acc, perm8)
            acc = acc + lane_perm(acc, perm4)
            acc = acc + lane_perm(acc, perm2)
            out_acc = jnp.where((lanes >> 1) == j, acc, out_acc)
        z = out_acc * (1.0 / HIST) + bias
        o_v[pl.ds(g * 16, 16)] = 1.0 / (1.0 + jnp.exp(-z))
        return carry

    lax.fori_loop(0, BAGS_PER_W // 8, per_group, 0)
    pltpu.sync_copy(o_v, out_hbm.at[pl.ds(wid * (BAGS_PER_W * NUM_Y),
                                          BAGS_PER_W * NUM_Y)])


def _sc_gather_pool(p_flat, idxd, bias16):
    mesh = plsc.VectorSubcoreMesh(core_axis_name="c", subcore_axis_name="s")
    kfn = functools.partial(
        pl.kernel,
        out_type=jax.ShapeDtypeStruct((BATCH * NUM_Y,), jnp.float32),
        mesh=mesh,
        scratch_types=[
            pltpu.VMEM((CHUNKS_PER_W, CHUNK), jnp.int32),
            pltpu.VMEM((VALS_PER_W,), jnp.float32),
            pltpu.VMEM((BAGS_PER_W * NUM_Y,), jnp.float32),
            pltpu.VMEM((16,), jnp.float32),
            pltpu.SemaphoreType.DMA,
        ],
        compiler_params=pltpu.CompilerParams(use_tc_tiling_on_sc=False),
    )(_sc_body)
    return kfn(p_flat, idxd, bias16)


def kernel(input, emb_weight, W, b):
    flat = input.astype(jnp.int32).reshape(-1)
    idxd = (flat[:, None] * 2 + jnp.arange(2, dtype=jnp.int32)[None, :]
            ).reshape(TOTAL_VALS // CHUNK, CHUNK)
    P = _project_table(emb_weight, W)
    bias16 = jnp.tile(b.astype(jnp.float32), 16 // NUM_Y)
    out = _sc_gather_pool(P.reshape(-1), idxd, bias16)
    return out.reshape(BATCH, NUM_Y)


# speedup vs baseline: 1.8074x; 1.8074x over previous
import functools

import jax
import jax.numpy as jnp
from jax import lax
from jax.experimental import pallas as pl
from jax.experimental.pallas import tpu as pltpu
from jax.experimental.pallas import tpu_sc as plsc

VOCAB = 1000000
EMB_DIM = 64
NUM_Y = 2
BATCH = 4096
HIST = 200

NC = 2
NS = 16
NW = NC * NS

TOTAL_IDX = BATCH * HIST
IDX_PER_W = TOTAL_IDX // NW
CHUNK = 128
CHUNKS_PER_W = IDX_PER_W // CHUNK
RING = 20
BAGS_PER_W = BATCH // NW
GROUPS_PER_W = BAGS_PER_W // 2

PROJ_BLK = 8000


def _proj_body(emb_ref, w_ref, out_hbm, pk_v, sem):
    i = pl.program_id(0)
    s = lax.dot_general(
        w_ref[...], emb_ref[...],
        (((1,), (1,)), ((), ())),
        preferred_element_type=jnp.float32,
    )
    s0 = s[0:1, :].astype(jnp.bfloat16)
    s1 = s[1:2, :].astype(jnp.bfloat16)
    u0 = lax.convert_element_type(
        lax.bitcast_convert_type(s0, jnp.uint16), jnp.uint32)
    u1 = lax.convert_element_type(
        lax.bitcast_convert_type(s1, jnp.uint16), jnp.uint32)
    packed = u0 | (u1 << 16)
    pk_v[...] = lax.bitcast_convert_type(packed, jnp.int32)
    cp = pltpu.make_async_copy(pk_v, out_hbm.at[pl.ds(i, 1), :], sem)
    cp.start()
    cp.wait()


def _project_pack(emb_weight, W):
    grid = VOCAB // PROJ_BLK
    return pl.pallas_call(
        _proj_body,
        grid=(grid,),
        in_specs=[
            pl.BlockSpec((PROJ_BLK, EMB_DIM), lambda i: (i, 0)),
            pl.BlockSpec((NUM_Y, EMB_DIM), lambda i: (0, 0)),
        ],
        out_specs=pl.BlockSpec(memory_space=pl.ANY),
        out_shape=jax.ShapeDtypeStruct((grid, PROJ_BLK), jnp.int32),
        scratch_shapes=[pltpu.VMEM((1, PROJ_BLK), jnp.int32),
                        pltpu.SemaphoreType.DMA],
    )(emb_weight, W)


def _sc_body(p_hbm, idx_hbm, bias_hbm, out_hbm, idx_v, g_v, o_v, b_v, sem):
    wid = lax.axis_index("s") * NC + lax.axis_index("c")
    pltpu.sync_copy(idx_hbm.at[pl.ds(wid * CHUNKS_PER_W, CHUNKS_PER_W)], idx_v)
    pltpu.sync_copy(bias_hbm, b_v)

    def chunk_copy(row):
        return pltpu.make_async_copy(
            p_hbm.at[idx_v.at[row]],
            g_v.at[pl.ds(row * CHUNK, CHUNK)],
            sem,
        )

    for j in range(RING):
        chunk_copy(j).start()

    def steady(r, carry):
        chunk_copy(r).wait()
        chunk_copy(r + RING).start()
        return carry

    lax.fori_loop(0, CHUNKS_PER_W - RING, steady, 0)
    for j in range(CHUNKS_PER_W - RING, CHUNKS_PER_W):
        chunk_copy(j).wait()

    lanes = lax.iota(jnp.int32, 16)
    bias = b_v[...]
    zero = jnp.zeros((16,), jnp.float32)
    himask = jnp.int32(-65536)
    dnums = lax.GatherDimensionNumbers(
        offset_dims=(), collapsed_slice_dims=(0,), start_index_map=(0,))
    perms = [(lanes + step) % 16 for step in (8, 4, 2, 1)]

    def lane_perm(x, perm):
        return lax.gather(x, perm[:, None], dnums, (1,),
                          mode=lax.GatherScatterMode.PROMISE_IN_BOUNDS)

    def fold(x):
        for p in perms:
            x = x + lane_perm(x, p)
        return x

    def unpack(w):
        lo = plsc.bitcast(w << 16, jnp.float32)
        hi = plsc.bitcast(w & himask, jnp.float32)
        return lo, hi

    def per_quad(q, carry):
        out_acc = zero
        for t4 in range(4):
            t = q * 4 + t4
            base = t * (2 * HIST)

            def redA(k, ab):
                a0, a1 = ab
                lo, hi = unpack(g_v[pl.ds(base + k * 16, 16)])
                return a0 + lo, a1 + hi

            accA0, accA1 = lax.fori_loop(0, 12, redA, (zero, zero))
            lo, hi = unpack(g_v[pl.ds(base + 192, 16)])
            mA = lanes < 8
            accA0 = accA0 + jnp.where(mA, lo, zero)
            accA1 = accA1 + jnp.where(mA, hi, zero)
            accB0 = jnp.where(mA, zero, lo)
            accB1 = jnp.where(mA, zero, hi)

            def redB(k, ab):
                a0, a1 = ab
                lo, hi = unpack(g_v[pl.ds(base + k * 16, 16)])
                return a0 + lo, a1 + hi

            accB0, accB1 = lax.fori_loop(13, 25, redB, (accB0, accB1))

            sA0, sA1 = fold(accA0), fold(accA1)
            sB0, sB1 = fold(accB0), fold(accB1)
            qA = jnp.where((lanes & 1) == 0, sA0, sA1)
            qB = jnp.where((lanes & 1) == 0, sB0, sB1)
            quadv = jnp.where((lanes & 2) == 0, qA, qB)
            out_acc = jnp.where((lanes >> 2) == t4, quadv, out_acc)
        z = out_acc * (1.0 / HIST) + bias
        o_v[pl.ds(q * 16, 16)] = 1.0 / (1.0 + jnp.exp(-z))
        return carry

    lax.fori_loop(0, GROUPS_PER_W // 4, per_quad, 0)
    pltpu.sync_copy(o_v, out_hbm.at[pl.ds(wid * (BAGS_PER_W * NUM_Y),
                                          BAGS_PER_W * NUM_Y)])


def _sc_gather_pool(p_flat, idx2, bias16):
    mesh = plsc.VectorSubcoreMesh(core_axis_name="c", subcore_axis_name="s")
    kfn = functools.partial(
        pl.kernel,
        out_type=jax.ShapeDtypeStruct((BATCH * NUM_Y,), jnp.float32),
        mesh=mesh,
        scratch_types=[
            pltpu.VMEM((CHUNKS_PER_W, CHUNK), jnp.int32),
            pltpu.VMEM((IDX_PER_W,), jnp.int32),
            pltpu.VMEM((BAGS_PER_W * NUM_Y,), jnp.float32),
            pltpu.VMEM((16,), jnp.float32),
            pltpu.SemaphoreType.DMA,
        ],
        compiler_params=pltpu.CompilerParams(use_tc_tiling_on_sc=False,
                                             needs_layout_passes=False),
    )(_sc_body)
    return kfn(p_flat, idx2, bias16)


def kernel(input, emb_weight, W, b):
    idx2 = input.astype(jnp.int32).reshape(TOTAL_IDX // CHUNK, CHUNK)
    Pp = _project_pack(emb_weight, W)
    bias16 = jnp.tile(b.astype(jnp.float32), 16 // NUM_Y)
    out = _sc_gather_pool(Pp.reshape(-1), idx2, bias16)
    return out.reshape(BATCH, NUM_Y)


# speedup vs baseline: 1.9528x; 1.0804x over previous
import functools

import jax
import jax.numpy as jnp
from jax import lax
from jax.experimental import pallas as pl
from jax.experimental.pallas import tpu as pltpu
from jax.experimental.pallas import tpu_sc as plsc

VOCAB = 1000000
EMB_DIM = 64
NUM_Y = 2
BATCH = 4096
HIST = 200

NC = 2
NS = 16
NW = NC * NS

TOTAL_IDX = BATCH * HIST
IDX_PER_W = TOTAL_IDX // NW
CHUNK = 128
CHUNKS_PER_W = IDX_PER_W // CHUNK
RING = 20
BAGS_PER_W = BATCH // NW
GROUPS_PER_W = BAGS_PER_W // 2

PROJ_BLK = 8192
PROJ_GRID = -(-VOCAB // PROJ_BLK)


def _proj_body(emb_ref, w_ref, out_hbm, pk_v, sems):
    i = pl.program_id(0)
    grid = PROJ_GRID
    slot = lax.rem(i, 2)
    s = lax.dot_general(
        w_ref[...], emb_ref[...],
        (((1,), (1,)), ((), ())),
        preferred_element_type=jnp.float32,
    )
    s0 = s[0:1, :].astype(jnp.bfloat16)
    s1 = s[1:2, :].astype(jnp.bfloat16)
    u0 = lax.convert_element_type(
        lax.bitcast_convert_type(s0, jnp.uint16), jnp.uint32)
    u1 = lax.convert_element_type(
        lax.bitcast_convert_type(s1, jnp.uint16), jnp.uint32)
    packed = u0 | (u1 << 16)
    pk_v[pl.ds(slot, 1), :] = lax.bitcast_convert_type(packed, jnp.int32)

    def cp(j, sl):
        return pltpu.make_async_copy(
            pk_v.at[sl], out_hbm.at[pl.ds(j * PROJ_BLK, PROJ_BLK)],
            sems.at[sl])

    cp(i, slot).start()

    @pl.when(i > 0)
    def _():
        cp(i - 1, 1 - slot).wait()

    @pl.when(i == grid - 1)
    def _():
        cp(i, slot).wait()


def _project_pack(emb_weight, W):
    grid = PROJ_GRID
    return pl.pallas_call(
        _proj_body,
        grid=(grid,),
        in_specs=[
            pl.BlockSpec((PROJ_BLK, EMB_DIM), lambda i: (i, 0)),
            pl.BlockSpec((NUM_Y, EMB_DIM), lambda i: (0, 0)),
        ],
        out_specs=pl.BlockSpec(memory_space=pl.ANY),
        out_shape=jax.ShapeDtypeStruct((PROJ_GRID * PROJ_BLK,), jnp.int32),
        scratch_shapes=[pltpu.VMEM((2, PROJ_BLK), jnp.int32),
                        pltpu.SemaphoreType.DMA((2,))],
    )(emb_weight, W)


def _sc_body(p_hbm, idx_hbm, bias_hbm, out_hbm, idx_v, g_v, o_v, b_v, sem):
    wid = lax.axis_index("s") * NC + lax.axis_index("c")
    pltpu.sync_copy(idx_hbm.at[pl.ds(wid * CHUNKS_PER_W, CHUNKS_PER_W)], idx_v)
    pltpu.sync_copy(bias_hbm, b_v)

    def chunk_copy(row):
        return pltpu.make_async_copy(
            p_hbm.at[idx_v.at[row]],
            g_v.at[pl.ds(row * CHUNK, CHUNK)],
            sem,
        )

    for j in range(RING):
        chunk_copy(j).start()

    def steady(r, carry):
        chunk_copy(r).wait()
        chunk_copy(r + RING).start()
        return carry

    lax.fori_loop(0, CHUNKS_PER_W - RING, steady, 0)
    for j in range(CHUNKS_PER_W - RING, CHUNKS_PER_W):
        chunk_copy(j).wait()

    lanes = lax.iota(jnp.int32, 16)
    bias = b_v[...]
    zero = jnp.zeros((16,), jnp.float32)
    himask = jnp.int32(-65536)
    dnums = lax.GatherDimensionNumbers(
        offset_dims=(), collapsed_slice_dims=(0,), start_index_map=(0,))
    perms = [(lanes + step) % 16 for step in (8, 4, 2, 1)]

    def lane_perm(x, perm):
        return lax.gather(x, perm[:, None], dnums, (1,),
                          mode=lax.GatherScatterMode.PROMISE_IN_BOUNDS)

    def fold(x):
        for p in perms:
            x = x + lane_perm(x, p)
        return x

    def unpack(w):
        lo = plsc.bitcast(w << 16, jnp.float32)
        hi = plsc.bitcast(w & himask, jnp.float32)
        return lo, hi

    def per_quad(q, carry):
        out_acc = zero
        for t4 in range(4):
            t = q * 4 + t4
            base = t * (2 * HIST)

            def redA(k, ab):
                a0, a1 = ab
                lo, hi = unpack(g_v[pl.ds(base + k * 16, 16)])
                return a0 + lo, a1 + hi

            accA0, accA1 = lax.fori_loop(0, 12, redA, (zero, zero))
            lo, hi = unpack(g_v[pl.ds(base + 192, 16)])
            mA = lanes < 8
            accA0 = accA0 + jnp.where(mA, lo, zero)
            accA1 = accA1 + jnp.where(mA, hi, zero)
            accB0 = jnp.where(mA, zero, lo)
            accB1 = jnp.where(mA, zero, hi)

            def redB(k, ab):
                a0, a1 = ab
                lo, hi = unpack(g_v[pl.ds(base + k * 16, 16)])
                return a0 + lo, a1 + hi

            accB0, accB1 = lax.fori_loop(13, 25, redB, (accB0, accB1))

            sA0, sA1 = fold(accA0), fold(accA1)
            sB0, sB1 = fold(accB0), fold(accB1)
            qA = jnp.where((lanes & 1) == 0, sA0, sA1)
            qB = jnp.where((lanes & 1) == 0, sB0, sB1)
            quadv = jnp.where((lanes & 2) == 0, qA, qB)
            out_acc = jnp.where((lanes >> 2) == t4, quadv, out_acc)
        z = out_acc * (1.0 / HIST) + bias
        o_v[pl.ds(q * 16, 16)] = 1.0 / (1.0 + jnp.exp(-z))
        return carry

    lax.fori_loop(0, GROUPS_PER_W // 4, per_quad, 0)
    pltpu.sync_copy(o_v, out_hbm.at[pl.ds(wid * (BAGS_PER_W * NUM_Y),
                                          BAGS_PER_W * NUM_Y)])


def _sc_gather_pool(p_flat, idx2, bias16):
    mesh = plsc.VectorSubcoreMesh(core_axis_name="c", subcore_axis_name="s")
    kfn = functools.partial(
        pl.kernel,
        out_type=jax.ShapeDtypeStruct((BATCH * NUM_Y,), jnp.float32),
        mesh=mesh,
        scratch_types=[
            pltpu.VMEM((CHUNKS_PER_W, CHUNK), jnp.int32),
            pltpu.VMEM((IDX_PER_W,), jnp.int32),
            pltpu.VMEM((BAGS_PER_W * NUM_Y,), jnp.float32),
            pltpu.VMEM((16,), jnp.float32),
            pltpu.SemaphoreType.DMA,
        ],
        compiler_params=pltpu.CompilerParams(use_tc_tiling_on_sc=False,
                                             needs_layout_passes=False),
    )(_sc_body)
    return kfn(p_flat, idx2, bias16)


def kernel(input, emb_weight, W, b):
    idx2 = input.astype(jnp.int32).reshape(TOTAL_IDX // CHUNK, CHUNK)
    Pp = _project_pack(emb_weight, W)
    bias16 = jnp.tile(b.astype(jnp.float32), 16 // NUM_Y)
    out = _sc_gather_pool(Pp, idx2, bias16)
    return out.reshape(BATCH, NUM_Y)


# speedup vs baseline: 5.7903x; 2.9652x over previous
import functools

import jax
import jax.numpy as jnp
from jax import lax
from jax.experimental import pallas as pl
from jax.experimental.pallas import tpu as pltpu
from jax.experimental.pallas import tpu_sc as plsc

VOCAB = 1000000
EMB_DIM = 64
NUM_Y = 2
BATCH = 4096
HIST = 200

NC = 2
NS = 16
NW = NC * NS

TOTAL_IDX = BATCH * HIST
IDX_PER_W = TOTAL_IDX // NW
CHUNK = 128
CHUNKS_PER_W = IDX_PER_W // CHUNK
RING = 20
BAGS_PER_W = BATCH // NW
GROUPS_PER_W = BAGS_PER_W // 2

PROJ_BLK = 8192
PROJ_GRID = -(-VOCAB // PROJ_BLK)


def _proj_body(embt_ref, w_ref, out_hbm, pk_v, sems):
    i = pl.program_id(0)
    grid = PROJ_GRID
    slot = lax.rem(i, 2)
    s = lax.dot_general(
        w_ref[...], embt_ref[...],
        (((1,), (0,)), ((), ())),
        preferred_element_type=jnp.float32,
    )
    s0 = s[0:1, :].astype(jnp.bfloat16)
    s1 = s[1:2, :].astype(jnp.bfloat16)
    u0 = lax.convert_element_type(
        lax.bitcast_convert_type(s0, jnp.uint16), jnp.uint32)
    u1 = lax.convert_element_type(
        lax.bitcast_convert_type(s1, jnp.uint16), jnp.uint32)
    packed = u0 | (u1 << 16)
    pk_v[pl.ds(slot, 1), :] = lax.bitcast_convert_type(packed, jnp.int32)

    def cp(j, sl):
        return pltpu.make_async_copy(
            pk_v.at[sl], out_hbm.at[pl.ds(j * PROJ_BLK, PROJ_BLK)],
            sems.at[sl])

    cp(i, slot).start()

    @pl.when(i > 0)
    def _():
        cp(i - 1, 1 - slot).wait()

    @pl.when(i == grid - 1)
    def _():
        cp(i, slot).wait()


def _project_pack(embT, W):
    grid = PROJ_GRID
    return pl.pallas_call(
        _proj_body,
        grid=(grid,),
        in_specs=[
            pl.BlockSpec((EMB_DIM, PROJ_BLK), lambda i: (0, i)),
            pl.BlockSpec((NUM_Y, EMB_DIM), lambda i: (0, 0)),
        ],
        out_specs=pl.BlockSpec(memory_space=pl.ANY),
        out_shape=jax.ShapeDtypeStruct((PROJ_GRID * PROJ_BLK,), jnp.int32),
        scratch_shapes=[pltpu.VMEM((2, PROJ_BLK), jnp.int32),
                        pltpu.SemaphoreType.DMA((2,))],
    )(embT, W)


def _sc_body(p_hbm, idx_hbm, bias_hbm, out_hbm, idx_v, g_v, o_v, b_v, sem):
    wid = lax.axis_index("s") * NC + lax.axis_index("c")
    pltpu.sync_copy(idx_hbm.at[pl.ds(wid * CHUNKS_PER_W, CHUNKS_PER_W)], idx_v)
    pltpu.sync_copy(bias_hbm, b_v)

    def chunk_copy(row):
        return pltpu.make_async_copy(
            p_hbm.at[idx_v.at[row]],
            g_v.at[pl.ds(row * CHUNK, CHUNK)],
            sem,
        )

    for j in range(RING):
        chunk_copy(j).start()

    def steady(r, carry):
        chunk_copy(r).wait()
        chunk_copy(r + RING).start()
        return carry

    lax.fori_loop(0, CHUNKS_PER_W - RING, steady, 0)
    for j in range(CHUNKS_PER_W - RING, CHUNKS_PER_W):
        chunk_copy(j).wait()

    lanes = lax.iota(jnp.int32, 16)
    bias = b_v[...]
    zero = jnp.zeros((16,), jnp.float32)
    himask = jnp.int32(-65536)
    dnums = lax.GatherDimensionNumbers(
        offset_dims=(), collapsed_slice_dims=(0,), start_index_map=(0,))
    perms = [(lanes + step) % 16 for step in (8, 4, 2, 1)]

    def lane_perm(x, perm):
        return lax.gather(x, perm[:, None], dnums, (1,),
                          mode=lax.GatherScatterMode.PROMISE_IN_BOUNDS)

    def fold(x):
        for p in perms:
            x = x + lane_perm(x, p)
        return x

    def unpack(w):
        lo = plsc.bitcast(w << 16, jnp.float32)
        hi = plsc.bitcast(w & himask, jnp.float32)
        return lo, hi

    def per_quad(q, carry):
        out_acc = zero
        for t4 in range(4):
            t = q * 4 + t4
            base = t * (2 * HIST)

            def redA(k, ab):
                a0, a1 = ab
                lo, hi = unpack(g_v[pl.ds(base + k * 16, 16)])
                return a0 + lo, a1 + hi

            accA0, accA1 = lax.fori_loop(0, 12, redA, (zero, zero))
            lo, hi = unpack(g_v[pl.ds(base + 192, 16)])
            mA = lanes < 8
            accA0 = accA0 + jnp.where(mA, lo, zero)
            accA1 = accA1 + jnp.where(mA, hi, zero)
            accB0 = jnp.where(mA, zero, lo)
            accB1 = jnp.where(mA, zero, hi)

            def redB(k, ab):
                a0, a1 = ab
                lo, hi = unpack(g_v[pl.ds(base + k * 16, 16)])
                return a0 + lo, a1 + hi

            accB0, accB1 = lax.fori_loop(13, 25, redB, (accB0, accB1))

            sA0, sA1 = fold(accA0), fold(accA1)
            sB0, sB1 = fold(accB0), fold(accB1)
            qA = jnp.where((lanes & 1) == 0, sA0, sA1)
            qB = jnp.where((lanes & 1) == 0, sB0, sB1)
            quadv = jnp.where((lanes & 2) == 0, qA, qB)
            out_acc = jnp.where((lanes >> 2) == t4, quadv, out_acc)
        z = out_acc * (1.0 / HIST) + bias
        o_v[pl.ds(q * 16, 16)] = 1.0 / (1.0 + jnp.exp(-z))
        return carry

    lax.fori_loop(0, GROUPS_PER_W // 4, per_quad, 0)
    pltpu.sync_copy(o_v, out_hbm.at[pl.ds(wid * (BAGS_PER_W * NUM_Y),
                                          BAGS_PER_W * NUM_Y)])


def _sc_gather_pool(p_flat, idx2, bias16):
    mesh = plsc.VectorSubcoreMesh(core_axis_name="c", subcore_axis_name="s")
    kfn = functools.partial(
        pl.kernel,
        out_type=jax.ShapeDtypeStruct((BATCH * NUM_Y,), jnp.float32),
        mesh=mesh,
        scratch_types=[
            pltpu.VMEM((CHUNKS_PER_W, CHUNK), jnp.int32),
            pltpu.VMEM((IDX_PER_W,), jnp.int32),
            pltpu.VMEM((BAGS_PER_W * NUM_Y,), jnp.float32),
            pltpu.VMEM((16,), jnp.float32),
            pltpu.SemaphoreType.DMA,
        ],
        compiler_params=pltpu.CompilerParams(use_tc_tiling_on_sc=False,
                                             needs_layout_passes=False),
    )(_sc_body)
    return kfn(p_flat, idx2, bias16)


def kernel(input, emb_weight, W, b):
    idx2 = input.astype(jnp.int32).reshape(TOTAL_IDX // CHUNK, CHUNK)
    Pp = _project_pack(emb_weight.T, W)
    bias16 = jnp.tile(b.astype(jnp.float32), 16 // NUM_Y)
    out = _sc_gather_pool(Pp, idx2, bias16)
    return out.reshape(BATCH, NUM_Y)


# speedup vs baseline: 7.2047x; 1.2443x over previous
import functools

import jax
import jax.numpy as jnp
from jax import lax
from jax.experimental import pallas as pl
from jax.experimental.pallas import tpu as pltpu
from jax.experimental.pallas import tpu_sc as plsc

VOCAB = 1000000
EMB_DIM = 64
NUM_Y = 2
BATCH = 4096
HIST = 200

NC = 2
NS = 16
NW = NC * NS

TOTAL_IDX = BATCH * HIST
IDX_PER_W = TOTAL_IDX // NW
CHUNK = 128
CHUNKS_PER_W = IDX_PER_W // CHUNK
RING = 40
BAGS_PER_W = BATCH // NW
GROUPS_PER_W = BAGS_PER_W // 2

PROJ_BLK = 16384
PROJ_GRID = -(-VOCAB // PROJ_BLK)


def _proj_body(embt_ref, w_ref, out_hbm, pk_v, sems):
    i = pl.program_id(0)
    grid = PROJ_GRID
    slot = lax.rem(i, 2)
    s = lax.dot_general(
        w_ref[...], embt_ref[...],
        (((1,), (0,)), ((), ())),
        preferred_element_type=jnp.float32,
    )
    s0 = s[0:1, :].astype(jnp.bfloat16)
    s1 = s[1:2, :].astype(jnp.bfloat16)
    u0 = lax.convert_element_type(
        lax.bitcast_convert_type(s0, jnp.uint16), jnp.uint32)
    u1 = lax.convert_element_type(
        lax.bitcast_convert_type(s1, jnp.uint16), jnp.uint32)
    packed = u0 | (u1 << 16)
    pk_v[pl.ds(slot, 1), :] = lax.bitcast_convert_type(packed, jnp.int32)

    def cp(j, sl):
        return pltpu.make_async_copy(
            pk_v.at[sl], out_hbm.at[pl.ds(j * PROJ_BLK, PROJ_BLK)],
            sems.at[sl])

    cp(i, slot).start()

    @pl.when(i > 0)
    def _():
        cp(i - 1, 1 - slot).wait()

    @pl.when(i == grid - 1)
    def _():
        cp(i, slot).wait()


def _project_pack(embT, W):
    grid = PROJ_GRID
    return pl.pallas_call(
        _proj_body,
        grid=(grid,),
        in_specs=[
            pl.BlockSpec((EMB_DIM, PROJ_BLK), lambda i: (0, i)),
            pl.BlockSpec((NUM_Y, EMB_DIM), lambda i: (0, 0)),
        ],
        out_specs=pl.BlockSpec(memory_space=pl.ANY),
        out_shape=jax.ShapeDtypeStruct((PROJ_GRID * PROJ_BLK,), jnp.int32),
        scratch_shapes=[pltpu.VMEM((2, PROJ_BLK), jnp.int32),
                        pltpu.SemaphoreType.DMA((2,))],
    )(embT, W)


def _sc_body(p_hbm, idx_hbm, bias_hbm, out_hbm, idx_v, g_v, o_v, b_v, sem):
    wid = lax.axis_index("s") * NC + lax.axis_index("c")
    pltpu.sync_copy(idx_hbm.at[pl.ds(wid * CHUNKS_PER_W, CHUNKS_PER_W)], idx_v)
    pltpu.sync_copy(bias_hbm, b_v)

    def chunk_copy(row):
        return pltpu.make_async_copy(
            p_hbm.at[idx_v.at[row]],
            g_v.at[pl.ds(row * CHUNK, CHUNK)],
            sem,
        )

    def prime(j, carry):
        chunk_copy(j).start()
        return carry

    lax.fori_loop(0, RING, prime, 0)

    def steady(r, carry):
        chunk_copy(r).wait()
        chunk_copy(r + RING).start()
        return carry

    lax.fori_loop(0, CHUNKS_PER_W - RING, steady, 0)

    def drain(j, carry):
        chunk_copy(j).wait()
        return carry

    lax.fori_loop(CHUNKS_PER_W - RING, CHUNKS_PER_W, drain, 0)

    lanes = lax.iota(jnp.int32, 16)
    bias = b_v[...]
    zero = jnp.zeros((16,), jnp.float32)
    himask = jnp.int32(-65536)
    dnums = lax.GatherDimensionNumbers(
        offset_dims=(), collapsed_slice_dims=(0,), start_index_map=(0,))
    perms = [(lanes + step) % 16 for step in (8, 4, 2, 1)]

    def lane_perm(x, perm):
        return lax.gather(x, perm[:, None], dnums, (1,),
                          mode=lax.GatherScatterMode.PROMISE_IN_BOUNDS)

    def fold(x):
        for p in perms:
            x = x + lane_perm(x, p)
        return x

    def unpack(w):
        lo = plsc.bitcast(w << 16, jnp.float32)
        hi = plsc.bitcast(w & himask, jnp.float32)
        return lo, hi

    def per_quad(q, carry):
        out_acc = zero
        for t4 in range(4):
            t = q * 4 + t4
            base = t * (2 * HIST)

            def redA(k, ab):
                a0, a1 = ab
                lo, hi = unpack(g_v[pl.ds(base + k * 16, 16)])
                return a0 + lo, a1 + hi

            accA0, accA1 = lax.fori_loop(0, 12, redA, (zero, zero))
            lo, hi = unpack(g_v[pl.ds(base + 192, 16)])
            mA = lanes < 8
            accA0 = accA0 + jnp.where(mA, lo, zero)
            accA1 = accA1 + jnp.where(mA, hi, zero)
            accB0 = jnp.where(mA, zero, lo)
            accB1 = jnp.where(mA, zero, hi)

            def redB(k, ab):
                a0, a1 = ab
                lo, hi = unpack(g_v[pl.ds(base + k * 16, 16)])
                return a0 + lo, a1 + hi

            accB0, accB1 = lax.fori_loop(13, 25, redB, (accB0, accB1))

            sA0, sA1 = fold(accA0), fold(accA1)
            sB0, sB1 = fold(accB0), fold(accB1)
            qA = jnp.where((lanes & 1) == 0, sA0, sA1)
            qB = jnp.where((lanes & 1) == 0, sB0, sB1)
            quadv = jnp.where((lanes & 2) == 0, qA, qB)
            out_acc = jnp.where((lanes >> 2) == t4, quadv, out_acc)
        z = out_acc * (1.0 / HIST) + bias
        o_v[pl.ds(q * 16, 16)] = 1.0 / (1.0 + jnp.exp(-z))
        return carry

    lax.fori_loop(0, GROUPS_PER_W // 4, per_quad, 0)
    pltpu.sync_copy(o_v, out_hbm.at[pl.ds(wid * (BAGS_PER_W * NUM_Y),
                                          BAGS_PER_W * NUM_Y)])


def _sc_gather_pool(p_flat, idx2, bias16):
    mesh = plsc.VectorSubcoreMesh(core_axis_name="c", subcore_axis_name="s")
    kfn = functools.partial(
        pl.kernel,
        out_type=jax.ShapeDtypeStruct((BATCH * NUM_Y,), jnp.float32),
        mesh=mesh,
        scratch_types=[
            pltpu.VMEM((CHUNKS_PER_W, CHUNK), jnp.int32),
            pltpu.VMEM((IDX_PER_W,), jnp.int32),
            pltpu.VMEM((BAGS_PER_W * NUM_Y,), jnp.float32),
            pltpu.VMEM((16,), jnp.float32),
            pltpu.SemaphoreType.DMA,
        ],
        compiler_params=pltpu.CompilerParams(use_tc_tiling_on_sc=False,
                                             needs_layout_passes=False),
    )(_sc_body)
    return kfn(p_flat, idx2, bias16)


def kernel(input, emb_weight, W, b):
    idx2 = input.astype(jnp.int32).reshape(TOTAL_IDX // CHUNK, CHUNK)
    Pp = _project_pack(emb_weight.T, W)
    bias16 = jnp.tile(b.astype(jnp.float32), 16 // NUM_Y)
    out = _sc_gather_pool(Pp, idx2, bias16)
    return out.reshape(BATCH, NUM_Y)
